# baseline (speedup 1.0000x reference)
import jax
import jax.numpy as jnp
from jax.experimental import pallas as pl
from jax.experimental.pallas import tpu as pltpu


def _fused_body(x_ref, w_ref, adj_ref, b_ref, out_ref, s_ref):
    @pl.when(pl.program_id(0) == 0)
    def _():
        s_ref[...] = jnp.dot(x_ref[...], w_ref[...],
                             preferred_element_type=jnp.float32
                             ).astype(jnp.bfloat16)

    acc = jnp.dot(adj_ref[...].astype(jnp.bfloat16), s_ref[...],
                  preferred_element_type=jnp.float32)
    out_ref[...] = acc + b_ref[...]


def kernel(input, adj, weight, bias):
    n, din = input.shape
    dout = weight.shape[1]

    bm = 400
    out = pl.pallas_call(
        _fused_body,
        grid=(n // bm,),
        in_specs=[
            pl.BlockSpec((n, din), lambda i: (0, 0)),
            pl.BlockSpec((din, dout), lambda i: (0, 0)),
            pl.BlockSpec((bm, n), lambda i: (i, 0)),
            pl.BlockSpec((1, dout), lambda i: (0, 0)),
        ],
        out_specs=pl.BlockSpec((bm, dout), lambda i: (i, 0)),
        out_shape=jax.ShapeDtypeStruct((n, dout), jnp.float32),
        scratch_shapes=[pltpu.VMEM((n, dout), jnp.bfloat16)],
        compiler_params=pltpu.CompilerParams(
            dimension_semantics=("arbitrary",),
            vmem_limit_bytes=100 * 1024 * 1024),
    )(input, weight, adj, bias.reshape(1, dout))
    return out


# speedup vs baseline: 1.0140x; 1.0140x over previous
import jax
import jax.numpy as jnp
from jax.experimental import pallas as pl
from jax.experimental.pallas import tpu as pltpu


def _fused_body(x_ref, w_ref, adj_ref, b_ref, out_ref, s_ref):
    @pl.when(pl.program_id(0) == 0)
    def _():
        s_ref[...] = jnp.dot(x_ref[...], w_ref[...],
                             preferred_element_type=jnp.float32)

    acc = jnp.dot(adj_ref[...], s_ref[...], preferred_element_type=jnp.float32)
    out_ref[...] = acc + b_ref[...]


def kernel(input, adj, weight, bias):
    n, din = input.shape
    dout = weight.shape[1]

    bm = 200
    out = pl.pallas_call(
        _fused_body,
        grid=(n // bm,),
        in_specs=[
            pl.BlockSpec((n, din), lambda i: (0, 0)),
            pl.BlockSpec((din, dout), lambda i: (0, 0)),
            pl.BlockSpec((bm, n), lambda i: (i, 0)),
            pl.BlockSpec((1, dout), lambda i: (0, 0)),
        ],
        out_specs=pl.BlockSpec((bm, dout), lambda i: (i, 0)),
        out_shape=jax.ShapeDtypeStruct((n, dout), jnp.float32),
        scratch_shapes=[pltpu.VMEM((n, dout), jnp.float32)],
        compiler_params=pltpu.CompilerParams(
            dimension_semantics=("arbitrary",),
            vmem_limit_bytes=100 * 1024 * 1024),
    )(input, weight, adj, bias.reshape(1, dout))
    return out


# speedup vs baseline: 1.0157x; 1.0016x over previous
import jax
import jax.numpy as jnp
from jax.experimental import pallas as pl
from jax.experimental.pallas import tpu as pltpu


def _fused_body(x_ref, w_ref, adj_ref, b_ref, out_ref, s_ref):
    @pl.when(pl.program_id(0) == 0)
    def _():
        s_ref[...] = jnp.dot(x_ref[...], w_ref[...],
                             preferred_element_type=jnp.float32)

    acc = jnp.dot(adj_ref[...], s_ref[...], preferred_element_type=jnp.float32)
    out_ref[...] = acc + b_ref[...]


def kernel(input, adj, weight, bias):
    n, din = input.shape
    dout = weight.shape[1]

    bm = 400
    out = pl.pallas_call(
        _fused_body,
        grid=(n // bm,),
        in_specs=[
            pl.BlockSpec((n, din), lambda i: (0, 0)),
            pl.BlockSpec((din, dout), lambda i: (0, 0)),
            pl.BlockSpec((bm, n), lambda i: (i, 0)),
            pl.BlockSpec((1, dout), lambda i: (0, 0)),
        ],
        out_specs=pl.BlockSpec((bm, dout), lambda i: (i, 0)),
        out_shape=jax.ShapeDtypeStruct((n, dout), jnp.float32),
        scratch_shapes=[pltpu.VMEM((n, dout), jnp.float32)],
        compiler_params=pltpu.CompilerParams(
            dimension_semantics=("arbitrary",),
            vmem_limit_bytes=100 * 1024 * 1024),
    )(input, weight, adj, bias.reshape(1, dout))
    return out
